# baseline (speedup 1.0000x reference)
import functools

import jax
import jax.numpy as jnp
from jax import lax
from jax.experimental import pallas as pl
from jax.experimental.pallas import tpu as pltpu
from jax.experimental.pallas import tpu_sc as plsc

_N = 100000
_C = 128
_B = 512
_T = 4

_BLK = 1024
_PG = 32
_NW = 32
_CH = 96
_NCH = 38
_RPW = _CH * _NCH
_NP = _NW * _RPW
_NBLK = _NP // _BLK
_W = 48
_BP = _B + _W


def _sc_gather(x, src2):
    mesh = plsc.VectorSubcoreMesh(core_axis_name="c", subcore_axis_name="s")

    @functools.partial(
        pl.kernel,
        out_type=jax.ShapeDtypeStruct((_NP, _C), jnp.float32),
        mesh=mesh,
        scratch_types=[
            pltpu.VMEM((_NCH, _CH), jnp.int32),
            pltpu.VMEM((_CH, _C), jnp.float32),
            pltpu.VMEM((_CH, _C), jnp.float32),
            pltpu.SemaphoreType.DMA,
            pltpu.SemaphoreType.DMA,
        ],
    )
    def gather_k(x_hbm, src_hbm, out_hbm, idx_v, buf0, buf1, sem0, sem1):
        wid = lax.axis_index("s") * 2 + lax.axis_index("c")
        base = wid * _RPW
        pltpu.sync_copy(src_hbm.at[wid], idx_v)

        def step(i, _):
            j0 = 2 * i
            j1 = 2 * i + 1
            cp0 = pltpu.async_copy(x_hbm.at[idx_v.at[j0]], buf0, sem0)
            cp1 = pltpu.async_copy(x_hbm.at[idx_v.at[j1]], buf1, sem1)
            cp0.wait()
            pltpu.sync_copy(buf0, out_hbm.at[pl.ds(base + j0 * _CH, _CH)])
            cp1.wait()
            pltpu.sync_copy(buf1, out_hbm.at[pl.ds(base + j1 * _CH, _CH)])
            return 0

        lax.fori_loop(0, _NCH // 2, step, 0)

    return gather_k(x, src2)


def _tc_body(bases_ref, x_ref, bat_ref, wih_ref, whh_ref, bias_ref, out_ref,
             qs_s, hp_s, c_s, m_s, d_s, s_s):
    t = pl.program_id(0)
    nb = pl.program_id(1)
    neg = jnp.float32(-jnp.inf)

    @pl.when(jnp.logical_and(t == 0, nb == 0))
    def _init():
        qs_s[...] = jnp.zeros_like(qs_s)
        hp_s[...] = jnp.zeros_like(hp_s)
        c_s[...] = jnp.zeros_like(c_s)

    @pl.when(nb == 0)
    def _lstm():
        gates = (
            jax.lax.dot_general(qs_s[...], wih_ref[...], (((1,), (1,)), ((), ())),
                                preferred_element_type=jnp.float32)
            + jax.lax.dot_general(hp_s[0:_B, :], whh_ref[...], (((1,), (1,)), ((), ())),
                                  preferred_element_type=jnp.float32)
            + bias_ref[...]
        )
        i_g = jax.nn.sigmoid(gates[:, 0 * _C:1 * _C])
        f_g = jax.nn.sigmoid(gates[:, 1 * _C:2 * _C])
        g_g = jnp.tanh(gates[:, 2 * _C:3 * _C])
        o_g = jax.nn.sigmoid(gates[:, 3 * _C:4 * _C])
        c = f_g * c_s[...] + i_g * g_g
        hp_s[0:_B, :] = o_g * jnp.tanh(c)
        c_s[...] = c
        m_s[...] = jnp.full_like(m_s, neg)
        d_s[...] = jnp.zeros_like(d_s)
        s_s[...] = jnp.zeros_like(s_s)

    base8 = pl.multiple_of((bases_ref[nb] // 8) * 8, 8)
    xb = x_ref[...]
    bat = bat_ref[0]

    seg = jax.lax.broadcasted_iota(jnp.int32, (_W, _BLK), 0) + base8
    onehot = seg == bat

    q_win = hp_s[pl.ds(base8, _W), :]
    xq = jax.lax.dot_general(q_win, xb, (((1,), (1,)), ((), ())),
                             preferred_element_type=jnp.float32)
    masked = jnp.where(onehot, xq, neg)
    m_old = m_s[pl.ds(base8, _W), :]
    m_new = jnp.maximum(m_old, jnp.max(masked, axis=1, keepdims=True))
    scale = jnp.where(m_old > neg, jnp.exp(m_old - m_new), 0.0)

    diff = jnp.sum(jnp.where(onehot, xq - m_new, 0.0), axis=0, keepdims=True)
    w_node = jnp.exp(diff)
    wmat = jnp.where(onehot, w_node, 0.0)

    d_s[pl.ds(base8, _W), :] = (d_s[pl.ds(base8, _W), :] * scale
                                + jnp.sum(wmat, axis=1, keepdims=True))
    s_s[pl.ds(base8, _W), :] = (s_s[pl.ds(base8, _W), :] * scale
                                + jax.lax.dot_general(
                                    wmat, xb, (((1,), (0,)), ((), ())),
                                    preferred_element_type=jnp.float32))
    m_s[pl.ds(base8, _W), :] = m_new

    @pl.when(nb == _NBLK - 1)
    def _finalize():
        r = s_s[0:_B, :] / (d_s[0:_B, :] + 1e-16)
        qs_s[...] = jnp.concatenate([hp_s[0:_B, :], r], axis=1)

    @pl.when(jnp.logical_and(t == _T - 1, nb == _NBLK - 1))
    def _emit():
        out_ref[...] = qs_s[...]


@jax.jit
def kernel(x, batch, W_ih, W_hh, b_ih, b_hh):
    bat = batch.astype(jnp.int32)

    b_all = jnp.arange(_B + 1, dtype=jnp.int32)
    starts_tbl = jnp.sum((bat[None, :] < b_all[:, None]).astype(jnp.int32), axis=1)
    counts = starts_tbl[1:] - starts_tbl[:-1]
    pn = ((counts + _PG - 1) // _PG) * _PG
    pstart = jnp.concatenate(
        [jnp.zeros((1,), jnp.int32), jnp.cumsum(pn)]).astype(jnp.int32)
    estart = pstart[:-1] + counts
    dshift = pstart[:-1] - starts_tbl[:-1]

    npg = _NP // _PG
    cpos = jnp.arange(npg, dtype=jnp.int32) * _PG
    elig = pstart[:-1][:, None] <= cpos[None, :]
    d_c = jnp.max(jnp.where(elig, dshift[:, None], 0), axis=0)
    e_c = jnp.max(jnp.where(elig, estart[:, None], 0), axis=0)
    seg_c = jnp.sum((pstart[1:][:, None] <= cpos[None, :]).astype(jnp.int32), axis=0)
    rep = lambda a: jnp.broadcast_to(a[:, None], (npg, _PG)).reshape(_NP)
    pos = jnp.arange(_NP, dtype=jnp.int32)
    is_real = pos < rep(e_c)
    src = jnp.clip(jnp.where(is_real, pos - rep(d_c), pos % _N), 0, _N - 1)
    bat_p = jnp.where(is_real, rep(seg_c), _B).astype(jnp.int32)
    bases = jnp.minimum(seg_c[::_BLK // _PG], _B - 1).astype(jnp.int32)

    xp = _sc_gather(x, src.reshape(_NW, _NCH, _CH))

    batp3 = bat_p.reshape(_NBLK, 1, _BLK)
    bias = (b_ih + b_hh).reshape(1, 4 * _C)

    grid_spec = pltpu.PrefetchScalarGridSpec(
        num_scalar_prefetch=1,
        grid=(_T, _NBLK),
        in_specs=[
            pl.BlockSpec((_BLK, _C), lambda t, nb, bases: (nb, 0)),
            pl.BlockSpec((1, 1, _BLK), lambda t, nb, bases: (nb, 0, 0)),
            pl.BlockSpec((4 * _C, 2 * _C), lambda t, nb, bases: (0, 0)),
            pl.BlockSpec((4 * _C, _C), lambda t, nb, bases: (0, 0)),
            pl.BlockSpec((1, 4 * _C), lambda t, nb, bases: (0, 0)),
        ],
        out_specs=pl.BlockSpec((_B, 2 * _C), lambda t, nb, bases: (0, 0)),
        scratch_shapes=[
            pltpu.VMEM((_B, 2 * _C), jnp.float32),
            pltpu.VMEM((_BP, _C), jnp.float32),
            pltpu.VMEM((_B, _C), jnp.float32),
            pltpu.VMEM((_BP, 1), jnp.float32),
            pltpu.VMEM((_BP, 1), jnp.float32),
            pltpu.VMEM((_BP, _C), jnp.float32),
        ],
    )
    return pl.pallas_call(
        _tc_body,
        grid_spec=grid_spec,
        out_shape=jax.ShapeDtypeStruct((_B, 2 * _C), jnp.float32),
        compiler_params=pltpu.CompilerParams(
            dimension_semantics=("arbitrary", "arbitrary"),
        ),
    )(bases, xp, batp3, W_ih, W_hh, bias)


# speedup vs baseline: 1.4099x; 1.4099x over previous
import functools

import jax
import jax.numpy as jnp
from jax import lax
from jax.experimental import pallas as pl
from jax.experimental.pallas import tpu as pltpu
from jax.experimental.pallas import tpu_sc as plsc

_N = 100000
_C = 128
_B = 512
_T = 4

_BLK = 2048
_PG = 32
_NW = 32
_CH = 96
_NCH = 38
_RPW = _CH * _NCH
_NP = _NW * _RPW
_NBLK = _NP // _BLK
_W = 80
_BP = _B + _W


def _sc_gather(x, src2):
    mesh = plsc.VectorSubcoreMesh(core_axis_name="c", subcore_axis_name="s")

    @functools.partial(
        pl.kernel,
        out_type=jax.ShapeDtypeStruct((_NP, _C), jnp.float32),
        mesh=mesh,
        scratch_types=[
            pltpu.VMEM((_NCH, _CH), jnp.int32),
            pltpu.VMEM((_CH, _C), jnp.float32),
            pltpu.VMEM((_CH, _C), jnp.float32),
            pltpu.SemaphoreType.DMA,
            pltpu.SemaphoreType.DMA,
        ],
    )
    def gather_k(x_hbm, src_hbm, out_hbm, idx_v, buf0, buf1, sem0, sem1):
        wid = lax.axis_index("s") * 2 + lax.axis_index("c")
        base = wid * _RPW
        pltpu.sync_copy(src_hbm.at[wid], idx_v)

        def step(i, _):
            j0 = 2 * i
            j1 = 2 * i + 1
            cp0 = pltpu.async_copy(x_hbm.at[idx_v.at[j0]], buf0, sem0)
            cp1 = pltpu.async_copy(x_hbm.at[idx_v.at[j1]], buf1, sem1)
            cp0.wait()
            pltpu.sync_copy(buf0, out_hbm.at[pl.ds(base + j0 * _CH, _CH)])
            cp1.wait()
            pltpu.sync_copy(buf1, out_hbm.at[pl.ds(base + j1 * _CH, _CH)])
            return 0

        lax.fori_loop(0, _NCH // 2, step, 0)

    return gather_k(x, src2)


def _tc_body(bases_ref, x_ref, bat_ref, wih_ref, whh_ref, bias_ref, out_ref,
             qs_s, hp_s, c_s, m_s, d_s, s_s):
    t = pl.program_id(0)
    nb = pl.program_id(1)
    neg = jnp.float32(-jnp.inf)

    @pl.when(jnp.logical_and(t == 0, nb == 0))
    def _init():
        qs_s[...] = jnp.zeros_like(qs_s)
        hp_s[...] = jnp.zeros_like(hp_s)
        c_s[...] = jnp.zeros_like(c_s)

    @pl.when(nb == 0)
    def _lstm():
        gates = (
            jax.lax.dot_general(qs_s[...], wih_ref[...], (((1,), (1,)), ((), ())),
                                preferred_element_type=jnp.float32)
            + jax.lax.dot_general(hp_s[0:_B, :], whh_ref[...], (((1,), (1,)), ((), ())),
                                  preferred_element_type=jnp.float32)
            + bias_ref[...]
        )
        i_g = jax.nn.sigmoid(gates[:, 0 * _C:1 * _C])
        f_g = jax.nn.sigmoid(gates[:, 1 * _C:2 * _C])
        g_g = jnp.tanh(gates[:, 2 * _C:3 * _C])
        o_g = jax.nn.sigmoid(gates[:, 3 * _C:4 * _C])
        c = f_g * c_s[...] + i_g * g_g
        hp_s[0:_B, :] = o_g * jnp.tanh(c)
        c_s[...] = c
        m_s[...] = jnp.full_like(m_s, neg)
        d_s[...] = jnp.zeros_like(d_s)
        s_s[...] = jnp.zeros_like(s_s)

    base8 = pl.multiple_of((bases_ref[nb] // 8) * 8, 8)
    xb = x_ref[...]
    bat = bat_ref[0]

    seg = jax.lax.broadcasted_iota(jnp.int32, (_W, _BLK), 0) + base8
    onehot = seg == bat

    q_win = hp_s[pl.ds(base8, _W), :]
    xq = jax.lax.dot_general(q_win, xb, (((1,), (1,)), ((), ())),
                             preferred_element_type=jnp.float32)
    masked = jnp.where(onehot, xq, neg)
    m_old = m_s[pl.ds(base8, _W), :]
    m_new = jnp.maximum(m_old, jnp.max(masked, axis=1, keepdims=True))
    scale = jnp.where(m_old > neg, jnp.exp(m_old - m_new), 0.0)

    diff = jnp.sum(jnp.where(onehot, xq - m_new, 0.0), axis=0, keepdims=True)
    w_node = jnp.exp(diff)
    wmat = jnp.where(onehot, w_node, 0.0)

    d_s[pl.ds(base8, _W), :] = (d_s[pl.ds(base8, _W), :] * scale
                                + jnp.sum(wmat, axis=1, keepdims=True))
    s_s[pl.ds(base8, _W), :] = (s_s[pl.ds(base8, _W), :] * scale
                                + jax.lax.dot_general(
                                    wmat, xb, (((1,), (0,)), ((), ())),
                                    preferred_element_type=jnp.float32))
    m_s[pl.ds(base8, _W), :] = m_new

    @pl.when(nb == _NBLK - 1)
    def _finalize():
        r = s_s[0:_B, :] / (d_s[0:_B, :] + 1e-16)
        qs_s[...] = jnp.concatenate([hp_s[0:_B, :], r], axis=1)

    @pl.when(jnp.logical_and(t == _T - 1, nb == _NBLK - 1))
    def _emit():
        out_ref[...] = qs_s[...]


@jax.jit
def kernel(x, batch, W_ih, W_hh, b_ih, b_hh):
    bat = batch.astype(jnp.int32)

    b_all = jnp.arange(_B + 1, dtype=jnp.int32)
    starts_tbl = jnp.sum((bat[None, :] < b_all[:, None]).astype(jnp.int32), axis=1)
    counts = starts_tbl[1:] - starts_tbl[:-1]
    pn = ((counts + _PG - 1) // _PG) * _PG
    pstart = jnp.concatenate(
        [jnp.zeros((1,), jnp.int32), jnp.cumsum(pn)]).astype(jnp.int32)
    estart = pstart[:-1] + counts
    dshift = pstart[:-1] - starts_tbl[:-1]

    npg = _NP // _PG
    cpos = jnp.arange(npg, dtype=jnp.int32) * _PG
    elig = pstart[:-1][:, None] <= cpos[None, :]
    d_c = jnp.max(jnp.where(elig, dshift[:, None], 0), axis=0)
    e_c = jnp.max(jnp.where(elig, estart[:, None], 0), axis=0)
    seg_c = jnp.sum((pstart[1:][:, None] <= cpos[None, :]).astype(jnp.int32), axis=0)
    rep = lambda a: jnp.broadcast_to(a[:, None], (npg, _PG)).reshape(_NP)
    pos = jnp.arange(_NP, dtype=jnp.int32)
    is_real = pos < rep(e_c)
    src = jnp.clip(jnp.where(is_real, pos - rep(d_c), pos % _N), 0, _N - 1)
    bat_p = jnp.where(is_real, rep(seg_c), _B).astype(jnp.int32)
    bases = jnp.minimum(seg_c[::_BLK // _PG], _B - 1).astype(jnp.int32)

    xp = _sc_gather(x, src.reshape(_NW, _NCH, _CH))

    batp3 = bat_p.reshape(_NBLK, 1, _BLK)
    bias = (b_ih + b_hh).reshape(1, 4 * _C)

    grid_spec = pltpu.PrefetchScalarGridSpec(
        num_scalar_prefetch=1,
        grid=(_T, _NBLK),
        in_specs=[
            pl.BlockSpec((_BLK, _C), lambda t, nb, bases: (nb, 0)),
            pl.BlockSpec((1, 1, _BLK), lambda t, nb, bases: (nb, 0, 0)),
            pl.BlockSpec((4 * _C, 2 * _C), lambda t, nb, bases: (0, 0)),
            pl.BlockSpec((4 * _C, _C), lambda t, nb, bases: (0, 0)),
            pl.BlockSpec((1, 4 * _C), lambda t, nb, bases: (0, 0)),
        ],
        out_specs=pl.BlockSpec((_B, 2 * _C), lambda t, nb, bases: (0, 0)),
        scratch_shapes=[
            pltpu.VMEM((_B, 2 * _C), jnp.float32),
            pltpu.VMEM((_BP, _C), jnp.float32),
            pltpu.VMEM((_B, _C), jnp.float32),
            pltpu.VMEM((_BP, 1), jnp.float32),
            pltpu.VMEM((_BP, 1), jnp.float32),
            pltpu.VMEM((_BP, _C), jnp.float32),
        ],
    )
    return pl.pallas_call(
        _tc_body,
        grid_spec=grid_spec,
        out_shape=jax.ShapeDtypeStruct((_B, 2 * _C), jnp.float32),
        compiler_params=pltpu.CompilerParams(
            dimension_semantics=("arbitrary", "arbitrary"),
        ),
    )(bases, xp, batp3, W_ih, W_hh, bias)
